# Initial kernel scaffold; baseline (speedup 1.0000x reference)
import jax
import jax.numpy as jnp
from jax.experimental import pallas as pl


def kernel(x, edge_index, self_kernel, neighbor_mlp_kernel, neighbor_mlp_bias, neighbor_kernel, bias):
    raise NotImplementedError("write your pallas kernel here")


# speedup vs baseline: 5.1992x; 5.1992x over previous
import functools

import jax
import jax.numpy as jnp
from jax import lax
from jax.experimental import pallas as pl
from jax.experimental.pallas import tpu as pltpu
from jax.experimental.pallas import tpu_sc as plsc

_HI = lax.Precision.HIGHEST

_NC = 2
_NS = 16
_NW = _NC * _NS
_K = 128
_G = 16
_W = 128


def _dense_body(x_ref, wmlp_ref, bmlp_ref, wn_ref, ws_ref, gw_ref, fx_ref):
    kv = wn_ref.shape[-1]
    x = x_ref[...]
    h = jnp.dot(x, wmlp_ref[...], precision=_HI) + bmlp_ref[...]
    h = jnp.maximum(h, 0.0)
    gw_ref[:, 0:kv] = jnp.dot(h, wn_ref[...], precision=_HI)
    gw_ref[:, kv:] = jnp.ones((x.shape[0], _W - kv), jnp.float32)
    fx_ref[...] = jnp.dot(x, ws_ref[...], precision=_HI)


def _combine_body(fx_ref, ps_ref, b_ref, o_ref):
    kv = fx_ref.shape[-1]
    ps = ps_ref[0] + ps_ref[1]
    s = ps[:, 0:kv]
    deg = ps[:, kv:kv + 1]
    mean = jnp.where(deg > 0.0, s / jnp.maximum(deg, 1.0), 0.0)
    o_ref[:, 0:kv] = jnp.maximum(fx_ref[...] + b_ref[0, 0:kv], 0.0)
    o_ref[:, kv:] = jnp.maximum(mean + b_ref[0, kv:], 0.0)


def _make_sc_agg(n_acc, n_groups):
    nps = n_acc // _NS
    wchunks = nps // _K
    mesh = plsc.VectorSubcoreMesh(core_axis_name="c", subcore_axis_name="s")
    out_type = jax.ShapeDtypeStruct((_NC, n_acc, _W), jnp.float32)
    scratch = [
        pltpu.VMEM((_G, _K), jnp.int32),
        pltpu.VMEM((_G, _K), jnp.int32),
        pltpu.VMEM((_K, _W), jnp.float32),
        pltpu.VMEM((_K, _W), jnp.float32),
        pltpu.VMEM_SHARED((n_acc, _W), jnp.float32),
        pltpu.SemaphoreType.DMA,
        pltpu.SemaphoreType.DMA,
    ]

    @functools.partial(pl.kernel, out_type=out_type, mesh=mesh,
                       scratch_types=scratch)
    def agg(gw_hbm, col_hbm, row_hbm, osum_hbm,
            colv, rowv, buf0, buf1, acc, sem0, sem1):
        cid = lax.axis_index("c")
        sid = lax.axis_index("s")
        wid = sid * _NC + cid
        base = sid * nps

        @pl.loop(0, _K)
        def _(i):
            for j in range(_W // 16):
                buf0[i, pl.ds(j * 16, 16)] = jnp.zeros((16,), jnp.float32)

        @pl.loop(0, wchunks)
        def _(j):
            pltpu.sync_copy(buf0, acc.at[pl.ds(base + j * _K, _K)])

        plsc.subcore_barrier()

        @pl.loop(0, n_groups)
        def _(grp):
            pltpu.sync_copy(col_hbm.at[wid].at[pl.ds(grp * _G, _G)], colv)
            pltpu.sync_copy(row_hbm.at[wid].at[pl.ds(grp * _G, _G)], rowv)

            pltpu.async_copy(gw_hbm.at[colv.at[0]], buf0, sem0)
            pltpu.async_copy(gw_hbm.at[colv.at[1]], buf1, sem1)

            @pl.loop(0, _G, step=2)
            def _(c):
                pltpu.make_async_copy(gw_hbm.at[colv.at[c]], buf0, sem0).wait()
                pltpu.sync_copy(buf0, acc.at[rowv.at[c]], add=True)

                @pl.when(c + 2 < _G)
                def _():
                    pltpu.async_copy(gw_hbm.at[colv.at[c + 2]], buf0, sem0)

                pltpu.make_async_copy(gw_hbm.at[colv.at[c + 1]], buf1, sem1).wait()
                pltpu.sync_copy(buf1, acc.at[rowv.at[c + 1]], add=True)

                @pl.when(c + 3 < _G)
                def _():
                    pltpu.async_copy(gw_hbm.at[colv.at[c + 3]], buf1, sem1)

        plsc.subcore_barrier()

        @pl.loop(0, wchunks)
        def _(j):
            pltpu.sync_copy(acc.at[pl.ds(base + j * _K, _K)], buf0)
            pltpu.sync_copy(buf0, osum_hbm.at[cid].at[pl.ds(base + j * _K, _K)])

    return agg


def kernel(x, edge_index, self_kernel, neighbor_mlp_kernel, neighbor_mlp_bias,
           neighbor_kernel, bias):
    n, d = x.shape
    e = edge_index.shape[1]
    h = neighbor_mlp_kernel.shape[1]
    kv = neighbor_kernel.shape[1]
    units = bias.shape[0]
    assert kv < _W

    step = _NW * _K * _G
    e_pad = -(-e // step) * step
    n_acc = -(-n // (_NS * _K)) * (_NS * _K)
    if e_pad > e and n_acc == n:
        n_acc += _NS * _K
    n_chunks = e_pad // (_NW * _K)
    n_groups = n_chunks // _G

    rb = 2000
    grid = (n // rb,)
    gw, fx = pl.pallas_call(
        _dense_body,
        grid=grid,
        in_specs=[
            pl.BlockSpec((rb, d), lambda i: (i, 0)),
            pl.BlockSpec((d, h), lambda i: (0, 0)),
            pl.BlockSpec((1, h), lambda i: (0, 0)),
            pl.BlockSpec((h, kv), lambda i: (0, 0)),
            pl.BlockSpec((d, kv), lambda i: (0, 0)),
        ],
        out_specs=[
            pl.BlockSpec((rb, _W), lambda i: (i, 0)),
            pl.BlockSpec((rb, kv), lambda i: (i, 0)),
        ],
        out_shape=[
            jax.ShapeDtypeStruct((n, _W), jnp.float32),
            jax.ShapeDtypeStruct((n, kv), jnp.float32),
        ],
    )(x, neighbor_mlp_kernel, neighbor_mlp_bias.reshape(1, h),
      neighbor_kernel, self_kernel)

    col = edge_index[1]
    row = edge_index[0]
    if e_pad > e:
        col = jnp.concatenate([col, jnp.zeros((e_pad - e,), col.dtype)])
        row = jnp.concatenate([row, jnp.full((e_pad - e,), n, row.dtype)])
    col3 = col.reshape(_NW, n_chunks, _K)
    row3 = row.reshape(_NW, n_chunks, _K)
    psum = _make_sc_agg(n_acc, n_groups)(gw, col3, row3)

    out = pl.pallas_call(
        _combine_body,
        grid=grid,
        in_specs=[
            pl.BlockSpec((rb, kv), lambda i: (i, 0)),
            pl.BlockSpec((_NC, rb, _W), lambda i: (0, i, 0)),
            pl.BlockSpec((1, units), lambda i: (0, 0)),
        ],
        out_specs=pl.BlockSpec((rb, units), lambda i: (i, 0)),
        out_shape=jax.ShapeDtypeStruct((n, units), jnp.float32),
    )(fx, psum, bias.reshape(1, units))
    return out


# speedup vs baseline: 7.2440x; 1.3933x over previous
import functools

import jax
import jax.numpy as jnp
from jax import lax
from jax.experimental import pallas as pl
from jax.experimental.pallas import tpu as pltpu
from jax.experimental.pallas import tpu_sc as plsc

_HI = lax.Precision.HIGHEST

_NC = 2
_NS = 16
_NW = _NC * _NS
_K = 128
_NB = 4
_DW = 16


def _dense_body(x_ref, wmlp_ref, bmlp_ref, wn_ref, ws_ref, g_ref, fx_ref):
    x = x_ref[...]
    h = jnp.dot(x, wmlp_ref[...], precision=_HI) + bmlp_ref[...]
    h = jnp.maximum(h, 0.0)
    g_ref[...] = jnp.dot(h, wn_ref[...], precision=_HI)
    fx_ref[...] = jnp.dot(x, ws_ref[...], precision=_HI)


def _combine_body(fx_ref, ps_ref, pd_ref, b_ref, o_ref):
    kv = fx_ref.shape[-1]
    s = ps_ref[0] + ps_ref[1]
    deg = pd_ref[0][:, 0:1] + pd_ref[1][:, 0:1]
    mean = jnp.where(deg > 0.0, s / jnp.maximum(deg, 1.0), 0.0)
    o_ref[:, 0:kv] = jnp.maximum(fx_ref[...] + b_ref[0, 0:kv], 0.0)
    o_ref[:, kv:] = jnp.maximum(mean + b_ref[0, kv:], 0.0)


def _make_sc_agg(n_acc, kv, n_chunks):
    nps = n_acc // _NS
    wchunks = nps // _K
    mesh = plsc.VectorSubcoreMesh(core_axis_name="c", subcore_axis_name="s")
    out_type = [
        jax.ShapeDtypeStruct((_NC, n_acc, kv), jnp.float32),
        jax.ShapeDtypeStruct((_NC, n_acc, _DW), jnp.float32),
    ]
    scratch = [
        pltpu.VMEM((n_chunks, _K), jnp.int32),
        pltpu.VMEM((n_chunks, _K), jnp.int32),
        [pltpu.VMEM((_K, kv), jnp.float32) for _ in range(_NB)],
        pltpu.VMEM((_K, _DW), jnp.float32),
        pltpu.VMEM_SHARED((n_acc, kv), jnp.float32),
        pltpu.VMEM_SHARED((n_acc, _DW), jnp.float32),
        [pltpu.SemaphoreType.DMA for _ in range(_NB)],
    ]

    @functools.partial(
        pl.kernel, out_type=out_type, mesh=mesh, scratch_types=scratch,
        compiler_params=pltpu.CompilerParams(use_tc_tiling_on_sc=False))
    def agg(g_hbm, col_hbm, row_hbm, osum_hbm, odeg_hbm,
            colv, rowv, bufs, small, acc, dacc, sems):
        cid = lax.axis_index("c")
        sid = lax.axis_index("s")
        wid = sid * _NC + cid
        base = sid * nps

        pltpu.sync_copy(col_hbm.at[wid], colv)
        pltpu.sync_copy(row_hbm.at[wid], rowv)

        @pl.loop(0, _K)
        def _(i):
            for j in range(kv // 16):
                bufs[0][i, pl.ds(j * 16, 16)] = jnp.zeros((16,), jnp.float32)
            small[i] = jnp.zeros((_DW,), jnp.float32)

        @pl.loop(0, wchunks)
        def _(j):
            pltpu.sync_copy(bufs[0], acc.at[pl.ds(base + j * _K, _K)])
            pltpu.sync_copy(small, dacc.at[pl.ds(base + j * _K, _K)])

        @pl.loop(0, _K)
        def _(i):
            small[i] = jnp.ones((_DW,), jnp.float32)

        plsc.subcore_barrier()

        for b in range(_NB):
            pltpu.async_copy(g_hbm.at[colv.at[b]], bufs[b], sems[b])

        @pl.loop(0, n_chunks, step=_NB)
        def _(c):
            for b in range(_NB):
                pltpu.make_async_copy(
                    g_hbm.at[colv.at[c + b]], bufs[b], sems[b]).wait()
                pltpu.sync_copy(bufs[b], acc.at[rowv.at[c + b]], add=True)
                pltpu.sync_copy(small, dacc.at[rowv.at[c + b]], add=True)

                @pl.when(c + b + _NB < n_chunks)
                def _():
                    pltpu.async_copy(
                        g_hbm.at[colv.at[c + b + _NB]], bufs[b], sems[b])

        plsc.subcore_barrier()

        @pl.loop(0, wchunks)
        def _(j):
            pltpu.sync_copy(acc.at[pl.ds(base + j * _K, _K)], bufs[0])
            pltpu.sync_copy(bufs[0], osum_hbm.at[cid].at[pl.ds(base + j * _K, _K)])
            pltpu.sync_copy(dacc.at[pl.ds(base + j * _K, _K)], small)
            pltpu.sync_copy(small, odeg_hbm.at[cid].at[pl.ds(base + j * _K, _K)])

    return agg


def kernel(x, edge_index, self_kernel, neighbor_mlp_kernel, neighbor_mlp_bias,
           neighbor_kernel, bias):
    n, d = x.shape
    e = edge_index.shape[1]
    h = neighbor_mlp_kernel.shape[1]
    kv = neighbor_kernel.shape[1]
    units = bias.shape[0]
    assert kv % 16 == 0

    step = _NW * _K * _NB
    e_pad = -(-e // step) * step
    n_acc = -(-n // (_NS * _K)) * (_NS * _K)
    if e_pad > e and n_acc == n:
        n_acc += _NS * _K
    n_chunks = e_pad // (_NW * _K)

    rb = 2000
    grid = (n // rb,)
    g, fx = pl.pallas_call(
        _dense_body,
        grid=grid,
        in_specs=[
            pl.BlockSpec((rb, d), lambda i: (i, 0)),
            pl.BlockSpec((d, h), lambda i: (0, 0)),
            pl.BlockSpec((1, h), lambda i: (0, 0)),
            pl.BlockSpec((h, kv), lambda i: (0, 0)),
            pl.BlockSpec((d, kv), lambda i: (0, 0)),
        ],
        out_specs=[
            pl.BlockSpec((rb, kv), lambda i: (i, 0)),
            pl.BlockSpec((rb, kv), lambda i: (i, 0)),
        ],
        out_shape=[
            jax.ShapeDtypeStruct((n, kv), jnp.float32),
            jax.ShapeDtypeStruct((n, kv), jnp.float32),
        ],
    )(x, neighbor_mlp_kernel, neighbor_mlp_bias.reshape(1, h),
      neighbor_kernel, self_kernel)

    col = edge_index[1]
    row = edge_index[0]
    if e_pad > e:
        col = jnp.concatenate([col, jnp.zeros((e_pad - e,), col.dtype)])
        row = jnp.concatenate([row, jnp.full((e_pad - e,), n, row.dtype)])
    col3 = col.reshape(_NW, n_chunks, _K)
    row3 = row.reshape(_NW, n_chunks, _K)
    psum, pdeg = _make_sc_agg(n_acc, kv, n_chunks)(g, col3, row3)

    out = pl.pallas_call(
        _combine_body,
        grid=grid,
        in_specs=[
            pl.BlockSpec((rb, kv), lambda i: (i, 0)),
            pl.BlockSpec((_NC, rb, kv), lambda i: (0, i, 0)),
            pl.BlockSpec((_NC, rb, _DW), lambda i: (0, i, 0)),
            pl.BlockSpec((1, units), lambda i: (0, 0)),
        ],
        out_specs=pl.BlockSpec((rb, units), lambda i: (i, 0)),
        out_shape=jax.ShapeDtypeStruct((n, units), jnp.float32),
    )(fx, psum, pdeg, bias.reshape(1, units))
    return out


# speedup vs baseline: 17.4779x; 2.4128x over previous
import functools

import jax
import jax.numpy as jnp
from jax import lax
from jax.experimental import pallas as pl
from jax.experimental.pallas import tpu as pltpu
from jax.experimental.pallas import tpu_sc as plsc

_HI = lax.Precision.HIGHEST

_NC = 2
_NS = 16
_NW = _NC * _NS
_K = 128
_NB = 4
_DW = 16


def _dense_body(x_ref, wmlp_ref, bmlp_ref, wn_ref, ws_ref, g_ref, fx_ref):
    x = x_ref[...]
    h = jnp.dot(x, wmlp_ref[...], precision=_HI) + bmlp_ref[...]
    h = jnp.maximum(h, 0.0)
    g_ref[...] = jnp.dot(h, wn_ref[...], precision=_HI)
    fx_ref[...] = jnp.dot(x, ws_ref[...], precision=_HI)


def _combine_body(fx_ref, ps_ref, pd_ref, b_ref, o_ref):
    kv = fx_ref.shape[-1]
    s = ps_ref[0] + ps_ref[1]
    deg = pd_ref[0][:, 0:1] + pd_ref[1][:, 0:1]
    mean = jnp.where(deg > 0.0, s / jnp.maximum(deg, 1.0), 0.0)
    o_ref[:, 0:kv] = jnp.maximum(fx_ref[...] + b_ref[0, 0:kv], 0.0)
    o_ref[:, kv:] = jnp.maximum(mean + b_ref[0, kv:], 0.0)


def _make_sc_agg(n_acc, kv, n_chunks):
    nps = n_acc // _NS
    wchunks = nps // _K
    mesh = plsc.VectorSubcoreMesh(core_axis_name="c", subcore_axis_name="s")
    out_type = [
        jax.ShapeDtypeStruct((_NC, n_acc, kv), jnp.float32),
        jax.ShapeDtypeStruct((_NC, n_acc, _DW), jnp.float32),
    ]
    scratch = [
        pltpu.VMEM((n_chunks, _K), jnp.int32),
        pltpu.VMEM((n_chunks, _K), jnp.int32),
        [pltpu.VMEM((_K, kv), jnp.float32) for _ in range(_NB)],
        pltpu.VMEM((_K, _DW), jnp.float32),
        pltpu.VMEM_SHARED((n_acc, kv), jnp.float32),
        pltpu.VMEM_SHARED((n_acc, _DW), jnp.float32),
        [pltpu.SemaphoreType.DMA for _ in range(_NB)],
    ]

    @functools.partial(
        pl.kernel, out_type=out_type, mesh=mesh, scratch_types=scratch,
        compiler_params=pltpu.CompilerParams(use_tc_tiling_on_sc=False))
    def agg(g_hbm, col_hbm, row_hbm, osum_hbm, odeg_hbm,
            colv, rowv, bufs, small, acc, dacc, sems):
        cid = lax.axis_index("c")
        sid = lax.axis_index("s")
        wid = sid * _NC + cid
        base = sid * nps

        pltpu.sync_copy(col_hbm.at[wid], colv)
        pltpu.sync_copy(row_hbm.at[wid], rowv)

        @pl.loop(0, _K)
        def _(i):
            for j in range(kv // 16):
                bufs[0][i, pl.ds(j * 16, 16)] = jnp.zeros((16,), jnp.float32)
            small[i] = jnp.zeros((_DW,), jnp.float32)

        @pl.loop(0, wchunks)
        def _(j):
            pltpu.sync_copy(bufs[0], acc.at[pl.ds(base + j * _K, _K)])
            pltpu.sync_copy(small, dacc.at[pl.ds(base + j * _K, _K)])

        @pl.loop(0, _K)
        def _(i):
            small[i] = jnp.ones((_DW,), jnp.float32)

        plsc.subcore_barrier()

        for b in range(_NB):
            pltpu.async_copy(g_hbm.at[colv.at[b]], bufs[b], sems[b])

        @pl.loop(0, n_chunks, step=_NB)
        def _(c):
            for b in range(_NB):
                pltpu.make_async_copy(
                    g_hbm.at[colv.at[c + b]], bufs[b], sems[b]).wait()
                pltpu.sync_copy(bufs[b], acc.at[rowv.at[c + b]], add=True)
                pltpu.sync_copy(small, dacc.at[rowv.at[c + b]], add=True)

                @pl.when(c + b + _NB < n_chunks)
                def _():
                    pltpu.async_copy(
                        g_hbm.at[colv.at[c + b + _NB]], bufs[b], sems[b])

        plsc.subcore_barrier()

        @pl.loop(0, wchunks)
        def _(j):
            pltpu.sync_copy(acc.at[pl.ds(base + j * _K, _K)], bufs[0])
            pltpu.sync_copy(bufs[0], osum_hbm.at[cid].at[pl.ds(base + j * _K, _K)])
            pltpu.sync_copy(dacc.at[pl.ds(base + j * _K, _K)], small)
            pltpu.sync_copy(small, odeg_hbm.at[cid].at[pl.ds(base + j * _K, _K)])

    return agg


def kernel(x, edge_index, self_kernel, neighbor_mlp_kernel, neighbor_mlp_bias,
           neighbor_kernel, bias):
    n, d = x.shape
    e = edge_index.shape[1]
    h = neighbor_mlp_kernel.shape[1]
    kv = neighbor_kernel.shape[1]
    units = bias.shape[0]
    assert kv % 16 == 0

    step = _NW * _K * _NB
    e_pad = -(-e // step) * step
    n_acc = -(-n // (_NS * _K)) * (_NS * _K)
    if e_pad > e and n_acc == n:
        n_acc += _NS * _K
    n_chunks = e_pad // (_NW * _K)

    rb = 2000
    grid = (n // rb,)
    g, fx = pl.pallas_call(
        _dense_body,
        grid=grid,
        in_specs=[
            pl.BlockSpec((rb, d), lambda i: (i, 0)),
            pl.BlockSpec((d, h), lambda i: (0, 0)),
            pl.BlockSpec((1, h), lambda i: (0, 0)),
            pl.BlockSpec((h, kv), lambda i: (0, 0)),
            pl.BlockSpec((d, kv), lambda i: (0, 0)),
        ],
        out_specs=[
            pl.BlockSpec((rb, kv), lambda i: (i, 0)),
            pl.BlockSpec((rb, kv), lambda i: (i, 0)),
        ],
        out_shape=[
            jax.ShapeDtypeStruct((n, kv), jnp.float32),
            jax.ShapeDtypeStruct((n, kv), jnp.float32),
        ],
    )(x, neighbor_mlp_kernel, neighbor_mlp_bias.reshape(1, h),
      neighbor_kernel, self_kernel)

    col = edge_index[1]
    row = edge_index[0]
    if e_pad > e:
        ar = jnp.arange(e_pad - e, dtype=col.dtype)
        col = jnp.concatenate([col, ar % n])
        row = jnp.concatenate([row, n + ar % (n_acc - n)])
    col3 = col.reshape(n_chunks, _NW, _K).transpose(1, 0, 2)
    row3 = row.reshape(n_chunks, _NW, _K).transpose(1, 0, 2)
    psum, pdeg = _make_sc_agg(n_acc, kv, n_chunks)(g, col3, row3)

    out = pl.pallas_call(
        _combine_body,
        grid=grid,
        in_specs=[
            pl.BlockSpec((rb, kv), lambda i: (i, 0)),
            pl.BlockSpec((_NC, rb, kv), lambda i: (0, i, 0)),
            pl.BlockSpec((_NC, rb, _DW), lambda i: (0, i, 0)),
            pl.BlockSpec((1, units), lambda i: (0, 0)),
        ],
        out_specs=pl.BlockSpec((rb, units), lambda i: (i, 0)),
        out_shape=jax.ShapeDtypeStruct((n, units), jnp.float32),
    )(fx, psum, pdeg, bias.reshape(1, units))
    return out


# speedup vs baseline: 20.5761x; 1.1773x over previous
import functools

import jax
import jax.numpy as jnp
from jax import lax
from jax.experimental import pallas as pl
from jax.experimental.pallas import tpu as pltpu
from jax.experimental.pallas import tpu_sc as plsc

_NC = 2
_NS = 16
_NW = _NC * _NS
_L = 16
_K = 128
_NB = 4


def _dense_body(x_ref, wmlp_ref, bmlp_ref, wn_ref, g_ref):
    h = jnp.dot(x_ref[...], wmlp_ref[...]) + bmlp_ref[...]
    h = jnp.maximum(h, 0.0)
    g_ref[...] = jnp.dot(h, wn_ref[...])


def _fx_body(x_ref, ws_ref, fx_ref):
    fx_ref[...] = jnp.dot(x_ref[...], ws_ref[...])


def _combine_body(fx_ref, ps_ref, pd_ref, b_ref, o_ref):
    kv = fx_ref.shape[-1]
    s = ps_ref[0] + ps_ref[1]
    deg = pd_ref[0] + pd_ref[1]
    mean = jnp.where(deg > 0.0, s / jnp.maximum(deg, 1.0), 0.0)
    o_ref[:, 0:kv] = jnp.maximum(fx_ref[...] + b_ref[0, 0:kv], 0.0)
    o_ref[:, kv:] = jnp.maximum(mean + b_ref[0, kv:], 0.0)


def _make_sc_agg(n_acc, kv, n_chunks):
    nps = n_acc // _NS
    wchunks = nps // _K
    ndr = n_acc // _L
    dch = ndr // _K
    dpt = ndr // _NS
    mesh = plsc.VectorSubcoreMesh(core_axis_name="c", subcore_axis_name="s")
    out_type = [
        jax.ShapeDtypeStruct((_NC, n_acc, kv), jnp.float32),
        jax.ShapeDtypeStruct((_NC, ndr, _L), jnp.float32),
    ]
    scratch = [
        pltpu.VMEM((n_chunks, _K), jnp.int32),
        pltpu.VMEM((n_chunks, _K), jnp.int32),
        [pltpu.VMEM((_K, kv), jnp.float32) for _ in range(_NB)],
        pltpu.VMEM((ndr, _L), jnp.float32),
        pltpu.VMEM((dch, _K), jnp.int32),
        pltpu.VMEM((dpt, _L), jnp.float32),
        pltpu.VMEM_SHARED((n_acc, kv), jnp.float32),
        pltpu.VMEM_SHARED((ndr, _L), jnp.float32),
        [pltpu.SemaphoreType.DMA for _ in range(_NB)],
    ]

    @functools.partial(
        pl.kernel, out_type=out_type, mesh=mesh, scratch_types=scratch,
        compiler_params=pltpu.CompilerParams(use_tc_tiling_on_sc=False,
                                             needs_layout_passes=False))
    def agg(g_hbm, col_hbm, row_hbm, osum_hbm, odeg_hbm,
            colv, rowv, bufs, degv, idxv, zbuf, acc, dacc, sems):
        cid = lax.axis_index("c")
        sid = lax.axis_index("s")
        wid = sid * _NC + cid
        base = sid * nps
        ones = jnp.ones((_L,), jnp.float32)

        pltpu.sync_copy(col_hbm.at[wid], colv)
        pltpu.sync_copy(row_hbm.at[wid], rowv)

        @pl.loop(0, ndr)
        def _(i):
            degv[i] = jnp.zeros((_L,), jnp.float32)

        @pl.loop(0, dch)
        def _(a):
            for b in range(_K // _L):
                idxv[a, pl.ds(b * _L, _L)] = (
                    a * _K + b * _L + lax.iota(jnp.int32, _L))

        @pl.loop(0, dpt)
        def _(i):
            zbuf[i] = jnp.zeros((_L,), jnp.float32)

        @pl.loop(0, _K)
        def _(i):
            for j in range(kv // _L):
                bufs[0][i, pl.ds(j * _L, _L)] = jnp.zeros((_L,), jnp.float32)

        @pl.loop(0, wchunks)
        def _(j):
            pltpu.sync_copy(bufs[0], acc.at[pl.ds(base + j * _K, _K)])

        pltpu.sync_copy(zbuf, dacc.at[pl.ds(sid * dpt, dpt)])

        plsc.subcore_barrier()

        for b in range(_NB):
            pltpu.async_copy(g_hbm.at[colv.at[b]], bufs[b], sems[b])

        @pl.loop(0, n_chunks, step=_NB)
        def _(c):
            for b in range(_NB):
                pltpu.make_async_copy(
                    g_hbm.at[colv.at[c + b]], bufs[b], sems[b]).wait()
                pltpu.sync_copy(bufs[b], acc.at[rowv.at[c + b]], add=True)

                @pl.when(c + b + _NB < n_chunks)
                def _():
                    pltpu.async_copy(
                        g_hbm.at[colv.at[c + b + _NB]], bufs[b], sems[b])

                for v in range(_K // _L):
                    r = rowv[c + b, pl.ds(v * _L, _L)]
                    plsc.addupdate_scatter(degv, [r >> 4, r & 15], ones)

        @pl.loop(0, dch)
        def _(a):
            pltpu.sync_copy(degv.at[pl.ds(a * _K, _K)],
                            dacc.at[idxv.at[a]], add=True)

        plsc.subcore_barrier()

        @pl.loop(0, wchunks)
        def _(j):
            pltpu.sync_copy(acc.at[pl.ds(base + j * _K, _K)], bufs[0])
            pltpu.sync_copy(bufs[0], osum_hbm.at[cid].at[pl.ds(base + j * _K, _K)])

        pltpu.sync_copy(dacc.at[pl.ds(sid * dpt, dpt)], zbuf)
        pltpu.sync_copy(zbuf, odeg_hbm.at[cid].at[pl.ds(sid * dpt, dpt)])

    return agg


def kernel(x, edge_index, self_kernel, neighbor_mlp_kernel, neighbor_mlp_bias,
           neighbor_kernel, bias):
    n, d = x.shape
    e = edge_index.shape[1]
    h = neighbor_mlp_kernel.shape[1]
    kv = neighbor_kernel.shape[1]
    units = bias.shape[0]
    assert kv % _L == 0

    step = _NW * _K * _NB
    e_pad = -(-e // step) * step
    n_acc = -(-n // (_NS * _K)) * (_NS * _K)
    if e_pad > e and n_acc == n:
        n_acc += _NS * _K
    n_chunks = e_pad // (_NW * _K)
    assert (n_acc // _L) % (_K * _NS) == 0 or (n_acc // _L) % _K == 0

    rb = 2000
    grid = (n // rb,)
    g = pl.pallas_call(
        _dense_body,
        grid=grid,
        in_specs=[
            pl.BlockSpec((rb, d), lambda i: (i, 0)),
            pl.BlockSpec((d, h), lambda i: (0, 0)),
            pl.BlockSpec((1, h), lambda i: (0, 0)),
            pl.BlockSpec((h, kv), lambda i: (0, 0)),
        ],
        out_specs=pl.BlockSpec((rb, kv), lambda i: (i, 0)),
        out_shape=jax.ShapeDtypeStruct((n, kv), jnp.float32),
    )(x, neighbor_mlp_kernel, neighbor_mlp_bias.reshape(1, h), neighbor_kernel)

    col = edge_index[1]
    row = edge_index[0]
    if e_pad > e:
        ar = jnp.arange(e_pad - e, dtype=col.dtype)
        col = jnp.concatenate([col, ar % n])
        row = jnp.concatenate([row, n + ar % (n_acc - n)])
    col3 = col.reshape(_NW, n_chunks, _K)
    row3 = row.reshape(_NW, n_chunks, _K)
    psum, pdeg = _make_sc_agg(n_acc, kv, n_chunks)(g, col3, row3)

    fx = pl.pallas_call(
        _fx_body,
        grid=grid,
        in_specs=[
            pl.BlockSpec((rb, d), lambda i: (i, 0)),
            pl.BlockSpec((d, kv), lambda i: (0, 0)),
        ],
        out_specs=pl.BlockSpec((rb, kv), lambda i: (i, 0)),
        out_shape=jax.ShapeDtypeStruct((n, kv), jnp.float32),
    )(x, self_kernel)

    pdeg3 = pdeg.reshape(_NC, n_acc, 1)
    out = pl.pallas_call(
        _combine_body,
        grid=grid,
        in_specs=[
            pl.BlockSpec((rb, kv), lambda i: (i, 0)),
            pl.BlockSpec((_NC, rb, kv), lambda i: (0, i, 0)),
            pl.BlockSpec((_NC, rb, 1), lambda i: (0, i, 0)),
            pl.BlockSpec((1, units), lambda i: (0, 0)),
        ],
        out_specs=pl.BlockSpec((rb, units), lambda i: (i, 0)),
        out_shape=jax.ShapeDtypeStruct((n, units), jnp.float32),
    )(fx, psum, pdeg3, bias.reshape(1, units))
    return out


# speedup vs baseline: 20.6201x; 1.0021x over previous
import functools

import jax
import jax.numpy as jnp
from jax import lax
from jax.experimental import pallas as pl
from jax.experimental.pallas import tpu as pltpu
from jax.experimental.pallas import tpu_sc as plsc

_NC = 2
_NS = 16
_NW = _NC * _NS
_L = 16
_K = 128
_NB = 4


def _dense_body(x_ref, wmlp_ref, bmlp_ref, wn_ref, g_ref):
    h = jnp.dot(x_ref[...], wmlp_ref[...]) + bmlp_ref[...]
    h = jnp.maximum(h, 0.0)
    g_ref[...] = jnp.dot(h, wn_ref[...])


def _fx_body(x_ref, ws_ref, fx_ref):
    fx_ref[...] = jnp.dot(x_ref[...], ws_ref[...])


def _combine_body(fx_ref, ps_ref, pd_ref, b_ref, o_ref):
    kv = fx_ref.shape[-1]
    s = ps_ref[0] + ps_ref[1]
    deg = pd_ref[0] + pd_ref[1]
    mean = jnp.where(deg > 0.0, s / jnp.maximum(deg, 1.0), 0.0)
    o_ref[:, 0:kv] = jnp.maximum(fx_ref[...] + b_ref[0, 0:kv], 0.0)
    o_ref[:, kv:] = jnp.maximum(mean + b_ref[0, kv:], 0.0)


def _make_sc_agg(n_acc, kv, n_chunks):
    nps = n_acc // _NS
    wchunks = nps // _K
    ndr = n_acc // _L
    dch = ndr // _K
    dpt = ndr // _NS
    mesh = plsc.VectorSubcoreMesh(core_axis_name="c", subcore_axis_name="s")
    out_type = [
        jax.ShapeDtypeStruct((_NC, n_acc, kv), jnp.float32),
        jax.ShapeDtypeStruct((_NC, ndr, _L), jnp.float32),
    ]
    epw = n_chunks * _K
    scratch = [
        pltpu.VMEM((epw,), jnp.int32),
        pltpu.VMEM((epw,), jnp.int32),
        [pltpu.VMEM((_K, kv), jnp.float32) for _ in range(_NB)],
        pltpu.VMEM((ndr, _L), jnp.float32),
        pltpu.VMEM((dch, _K), jnp.int32),
        pltpu.VMEM((dpt, _L), jnp.float32),
        pltpu.VMEM_SHARED((n_acc, kv), jnp.float32),
        pltpu.VMEM_SHARED((ndr, _L), jnp.float32),
        [pltpu.SemaphoreType.DMA for _ in range(_NB)],
    ]

    @functools.partial(
        pl.kernel, out_type=out_type, mesh=mesh, scratch_types=scratch,
        compiler_params=pltpu.CompilerParams(use_tc_tiling_on_sc=False,
                                             needs_layout_passes=False))
    def agg(g_hbm, col_hbm, row_hbm, osum_hbm, odeg_hbm,
            colv, rowv, bufs, degv, idxv, zbuf, acc, dacc, sems):
        cid = lax.axis_index("c")
        sid = lax.axis_index("s")
        wid = sid * _NC + cid
        base = sid * nps
        ones = jnp.ones((_L,), jnp.float32)

        pltpu.sync_copy(col_hbm.at[pl.ds(wid * epw, epw)], colv)
        pltpu.sync_copy(row_hbm.at[pl.ds(wid * epw, epw)], rowv)

        @pl.loop(0, ndr)
        def _(i):
            degv[i] = jnp.zeros((_L,), jnp.float32)

        @pl.loop(0, dch)
        def _(a):
            for b in range(_K // _L):
                idxv[a, pl.ds(b * _L, _L)] = (
                    a * _K + b * _L + lax.iota(jnp.int32, _L))

        @pl.loop(0, dpt)
        def _(i):
            zbuf[i] = jnp.zeros((_L,), jnp.float32)

        @pl.loop(0, _K)
        def _(i):
            for j in range(kv // _L):
                bufs[0][i, pl.ds(j * _L, _L)] = jnp.zeros((_L,), jnp.float32)

        @pl.loop(0, wchunks)
        def _(j):
            pltpu.sync_copy(bufs[0], acc.at[pl.ds(base + j * _K, _K)])

        pltpu.sync_copy(zbuf, dacc.at[pl.ds(sid * dpt, dpt)])

        plsc.subcore_barrier()

        for b in range(_NB):
            pltpu.async_copy(g_hbm.at[colv.at[pl.ds(b * _K, _K)]],
                             bufs[b], sems[b])

        @pl.loop(0, n_chunks, step=_NB)
        def _(c):
            for b in range(_NB):
                cb = (c + b) * _K
                pltpu.make_async_copy(
                    g_hbm.at[colv.at[pl.ds(cb, _K)]], bufs[b], sems[b]).wait()
                pltpu.sync_copy(bufs[b], acc.at[rowv.at[pl.ds(cb, _K)]],
                                add=True)

                @pl.when(c + b + _NB < n_chunks)
                def _():
                    pltpu.async_copy(
                        g_hbm.at[colv.at[pl.ds(cb + _NB * _K, _K)]],
                        bufs[b], sems[b])

                for v in range(_K // _L):
                    r = rowv[pl.ds(cb + v * _L, _L)]
                    plsc.addupdate_scatter(degv, [r >> 4, r & 15], ones)

        @pl.loop(0, dch)
        def _(a):
            pltpu.sync_copy(degv.at[pl.ds(a * _K, _K)],
                            dacc.at[idxv.at[a]], add=True)

        plsc.subcore_barrier()

        @pl.loop(0, wchunks)
        def _(j):
            pltpu.sync_copy(acc.at[pl.ds(base + j * _K, _K)], bufs[0])
            pltpu.sync_copy(bufs[0], osum_hbm.at[cid].at[pl.ds(base + j * _K, _K)])

        pltpu.sync_copy(dacc.at[pl.ds(sid * dpt, dpt)], zbuf)
        pltpu.sync_copy(zbuf, odeg_hbm.at[cid].at[pl.ds(sid * dpt, dpt)])

    return agg


def kernel(x, edge_index, self_kernel, neighbor_mlp_kernel, neighbor_mlp_bias,
           neighbor_kernel, bias):
    n, d = x.shape
    e = edge_index.shape[1]
    h = neighbor_mlp_kernel.shape[1]
    kv = neighbor_kernel.shape[1]
    units = bias.shape[0]
    assert kv % _L == 0

    step = _NW * _K * _NB
    e_pad = -(-e // step) * step
    n_acc = -(-n // (_NS * _K)) * (_NS * _K)
    if e_pad > e and n_acc == n:
        n_acc += _NS * _K
    n_chunks = e_pad // (_NW * _K)
    assert (n_acc // _L) % (_K * _NS) == 0 or (n_acc // _L) % _K == 0

    rb = 2000
    grid = (n // rb,)
    g = pl.pallas_call(
        _dense_body,
        grid=grid,
        in_specs=[
            pl.BlockSpec((rb, d), lambda i: (i, 0)),
            pl.BlockSpec((d, h), lambda i: (0, 0)),
            pl.BlockSpec((1, h), lambda i: (0, 0)),
            pl.BlockSpec((h, kv), lambda i: (0, 0)),
        ],
        out_specs=pl.BlockSpec((rb, kv), lambda i: (i, 0)),
        out_shape=jax.ShapeDtypeStruct((n, kv), jnp.float32),
    )(x, neighbor_mlp_kernel, neighbor_mlp_bias.reshape(1, h), neighbor_kernel)

    col = edge_index[1]
    row = edge_index[0]
    if e_pad > e:
        ar = jnp.arange(e_pad - e, dtype=col.dtype)
        col = jnp.concatenate([col, ar % n])
        row = jnp.concatenate([row, n + ar % (n_acc - n)])
    psum, pdeg = _make_sc_agg(n_acc, kv, n_chunks)(g, col, row)

    fx = pl.pallas_call(
        _fx_body,
        grid=grid,
        in_specs=[
            pl.BlockSpec((rb, d), lambda i: (i, 0)),
            pl.BlockSpec((d, kv), lambda i: (0, 0)),
        ],
        out_specs=pl.BlockSpec((rb, kv), lambda i: (i, 0)),
        out_shape=jax.ShapeDtypeStruct((n, kv), jnp.float32),
    )(x, self_kernel)

    pdeg3 = pdeg.reshape(_NC, n_acc, 1)
    out = pl.pallas_call(
        _combine_body,
        grid=grid,
        in_specs=[
            pl.BlockSpec((rb, kv), lambda i: (i, 0)),
            pl.BlockSpec((_NC, rb, kv), lambda i: (0, i, 0)),
            pl.BlockSpec((_NC, rb, 1), lambda i: (0, i, 0)),
            pl.BlockSpec((1, units), lambda i: (0, 0)),
        ],
        out_specs=pl.BlockSpec((rb, units), lambda i: (i, 0)),
        out_shape=jax.ShapeDtypeStruct((n, units), jnp.float32),
    )(fx, psum, pdeg3, bias.reshape(1, units))
    return out
